# Initial kernel scaffold; baseline (speedup 1.0000x reference)
import jax
import jax.numpy as jnp
from jax.experimental import pallas as pl


def kernel(x, geometry):
    raise NotImplementedError("write your pallas kernel here")


# speedup vs baseline: 5.8029x; 5.8029x over previous
import functools

import jax
import jax.numpy as jnp
from jax import lax
from jax.experimental import pallas as pl
from jax.experimental.pallas import tpu as pltpu
from jax.experimental.pallas import tpu_sc as plsc

BATCH = 16384
HIST = 50
EMBED = 64
B = BATCH * HIST

_NC = 2
_NS = 16
NW = _NC * _NS
B_PER_W = B // NW
CHUNK = 512
NCHUNK = B_PER_W // CHUNK


def _gather_kernel(table_hbm, idx_hbm, out_hbm, idx_v, rows_v, sem):
    wid = lax.axis_index("s") * _NC + lax.axis_index("c")
    base = wid * B_PER_W

    def body(i, carry):
        off = base + i * CHUNK
        pltpu.sync_copy(idx_hbm.at[pl.ds(off, CHUNK)], idx_v)
        pltpu.async_copy(table_hbm.at[idx_v], rows_v, sem).wait()
        pltpu.sync_copy(rows_v, out_hbm.at[pl.ds(off, CHUNK)])
        return carry

    lax.fori_loop(0, NCHUNK, body, 0)


def kernel(x, geometry):
    idx = x.reshape(B)
    mesh = plsc.VectorSubcoreMesh(core_axis_name="c", subcore_axis_name="s")
    run = functools.partial(
        pl.kernel,
        mesh=mesh,
        out_type=jax.ShapeDtypeStruct((B, EMBED), jnp.float32),
        scratch_types=[
            pltpu.VMEM((CHUNK,), jnp.int32),
            pltpu.VMEM((CHUNK, EMBED), jnp.float32),
            pltpu.SemaphoreType.DMA,
        ],
        compiler_params=pltpu.CompilerParams(use_tc_tiling_on_sc=False),
    )(_gather_kernel)
    out = run(geometry, idx)
    return out.reshape(BATCH, HIST, EMBED)


# speedup vs baseline: 6.2604x; 1.0788x over previous
import functools

import jax
import jax.numpy as jnp
from jax import lax
from jax.experimental import pallas as pl
from jax.experimental.pallas import tpu as pltpu
from jax.experimental.pallas import tpu_sc as plsc

BATCH = 16384
HIST = 50
EMBED = 64
B = BATCH * HIST

_NC = 2
_NS = 16
NW = _NC * _NS
B_PER_W = B // NW
CHUNK = 512
NCHUNK = B_PER_W // CHUNK
NBUF = 2


def _gather_kernel(table_hbm, idx_hbm, out_hbm,
                   idx_v, buf0, buf1, sg0, sg1, ss0, ss1):
    wid = lax.axis_index("s") * _NC + lax.axis_index("c")
    base = wid * B_PER_W

    bufs = (buf0, buf1)
    gsems = (sg0, sg1)
    ssems = (ss0, ss1)

    pltpu.sync_copy(idx_hbm.at[pl.ds(base, B_PER_W)], idx_v)

    def idx_slice(i):
        return idx_v.at[pl.ds(i * CHUNK, CHUNK)]

    def out_slice(i):
        return out_hbm.at[pl.ds(base + i * CHUNK, CHUNK)]

    for b in range(NBUF):
        pltpu.async_copy(table_hbm.at[idx_slice(b)], bufs[b], gsems[b])

    def body(i0, carry):
        for b in range(NBUF):
            i = i0 + b
            pltpu.make_async_copy(table_hbm.at[idx_slice(i)],
                                  bufs[b], gsems[b]).wait()
            pltpu.async_copy(bufs[b], out_slice(i), ssems[b])

            @pl.when(i + NBUF < NCHUNK)
            def _():
                pltpu.make_async_copy(bufs[b], out_slice(i), ssems[b]).wait()
                pltpu.async_copy(table_hbm.at[idx_slice(i + NBUF)],
                                 bufs[b], gsems[b])

        return carry

    lax.fori_loop(0, NCHUNK // NBUF, lambda g, c: body(g * NBUF, c), 0,
                  unroll=False)

    for b in range(NBUF):
        i = NCHUNK - NBUF + b
        pltpu.make_async_copy(bufs[b], out_slice(i), ssems[b]).wait()


def kernel(x, geometry):
    idx = x.reshape(B)
    mesh = plsc.VectorSubcoreMesh(core_axis_name="c", subcore_axis_name="s")
    run = functools.partial(
        pl.kernel,
        mesh=mesh,
        out_type=jax.ShapeDtypeStruct((B, EMBED), jnp.float32),
        scratch_types=[
            pltpu.VMEM((B_PER_W,), jnp.int32),
            pltpu.VMEM((CHUNK, EMBED), jnp.float32),
            pltpu.VMEM((CHUNK, EMBED), jnp.float32),
            pltpu.SemaphoreType.DMA,
            pltpu.SemaphoreType.DMA,
            pltpu.SemaphoreType.DMA,
            pltpu.SemaphoreType.DMA,
        ],
        compiler_params=pltpu.CompilerParams(use_tc_tiling_on_sc=False),
    )(_gather_kernel)
    out = run(geometry, idx)
    return out.reshape(BATCH, HIST, EMBED)
